# baseline (speedup 1.0000x reference)
import functools

import jax
import jax.numpy as jnp
from jax import lax
from jax.experimental import pallas as pl
from jax.experimental.pallas import tpu as pltpu
from jax.experimental.pallas import tpu_sc as plsc

BATCH = 4096
SEQ = 200
EMBED = 32
VOCAB = 1000000
N = BATCH * SEQ
NC = 2
NS = 16
NW = NC * NS

_mesh = plsc.VectorSubcoreMesh(core_axis_name="c", subcore_axis_name="s")


def _iota16():
    return lax.iota(jnp.int32, 16)


K1_C = 512
K1_MAIN = (VOCAB // 128) * 128
K1_CHUNKS = K1_MAIN // K1_C
K1_TAIL = VOCAB - K1_MAIN


@functools.partial(
    pl.kernel,
    mesh=_mesh,
    out_type=jax.ShapeDtypeStruct((VOCAB // 4, 128), jnp.float32),
    scratch_types=[
        pltpu.VMEM((EMBED, K1_C), jnp.float32),
        pltpu.VMEM((EMBED, K1_C), jnp.float32),
        pltpu.VMEM((K1_C // 4, 128), jnp.float32),
        pltpu.VMEM((K1_C // 4, 128), jnp.float32),
        pltpu.VMEM((EMBED, 64), jnp.float32),
        pltpu.SemaphoreType.DMA,
        pltpu.SemaphoreType.DMA,
        pltpu.SemaphoreType.DMA,
        pltpu.SemaphoreType.DMA,
    ],
    compiler_params=pltpu.CompilerParams(
        use_tc_tiling_on_sc=True, needs_layout_passes=False
    ),
)
def _relayout_kernel(tt, tail, t128, in0, in1, out0, out1, tin,
                     si0, si1, so0, so1):
    wid = lax.axis_index("s") * NC + lax.axis_index("c")
    nchunk = K1_CHUNKS // NW
    rem = K1_CHUNKS - nchunk * NW
    my_total = nchunk + jnp.where(wid < rem, 1, 0)
    e_lo = _iota16()
    e_hi = e_lo + 16

    ins = (in0, in1)
    outs = (out0, out1)
    sis = (si0, si1)
    sos = (so0, so1)

    def shuffle(in_buf, out_buf):
        def body(rr, _):
            q = rr // 4
            c0 = (rr % 4) * 32
            lo = plsc.load_gather(in_buf, [e_lo, jnp.full((16,), rr, jnp.int32)])
            hi = plsc.load_gather(in_buf, [e_hi, jnp.full((16,), rr, jnp.int32)])
            out_buf[q, pl.ds(c0, 16)] = lo
            out_buf[q, pl.ds(c0 + 16, 16)] = hi
            return 0
        lax.fori_loop(0, K1_C, body, 0, unroll=8)

    def chunk_r0(k):
        return pl.multiple_of((k * NW + wid) * K1_C, K1_C)

    def start_in(k, slot):
        return pltpu.async_copy(tt.at[:, pl.ds(chunk_r0(k), K1_C)],
                                ins[slot], sis[slot])

    def wait_in(slot):
        pltpu.make_async_copy(tt.at[:, pl.ds(0, K1_C)], ins[slot],
                              sis[slot]).wait()

    def wait_out(slot):
        pltpu.make_async_copy(outs[slot], t128.at[pl.ds(0, K1_C // 4), :],
                              sos[slot]).wait()

    start_in(0, 0)
    npair = (nchunk + 2) // 2

    def pair(p, _):
        for b in range(2):
            k = p * 2 + b
            @pl.when(k + 1 < my_total)
            def _():
                start_in(k + 1, 1 - b)
            @pl.when((k >= 2) & (k < my_total))
            def _():
                wait_out(b)
            @pl.when(k < my_total)
            def _():
                wait_in(b)
                shuffle(ins[b], outs[b])
                q0 = pl.multiple_of(chunk_r0(k) // 4, K1_C // 4)
                pltpu.async_copy(outs[b], t128.at[pl.ds(q0, K1_C // 4), :],
                                 sos[b])
        return 0

    lax.fori_loop(0, npair, pair, 0)
    wait_out(0)
    wait_out(1)

    @pl.when(wid == NW - 1)
    def _():
        pltpu.sync_copy(tail, tin)
        def body(rr, _):
            q = rr // 4
            c0 = (rr % 4) * 32
            lo = plsc.load_gather(tin, [e_lo, jnp.full((16,), rr, jnp.int32)])
            hi = plsc.load_gather(tin, [e_hi, jnp.full((16,), rr, jnp.int32)])
            out0[q, pl.ds(c0, 16)] = lo
            out0[q, pl.ds(c0 + 16, 16)] = hi
            return 0
        lax.fori_loop(0, K1_TAIL, body, 0, unroll=8)
        pltpu.sync_copy(out0.at[pl.ds(0, K1_TAIL // 4), :],
                        t128.at[pl.ds(K1_MAIN // 4, K1_TAIL // 4), :])


B_PER_W = BATCH // NW
SR = 8
NCH2 = SEQ // SR
GS = SR * B_PER_W


@functools.partial(
    pl.kernel,
    mesh=_mesh,
    out_type=jax.ShapeDtypeStruct((SEQ, 128, 1024), jnp.float32),
    scratch_types=[
        pltpu.VMEM((B_PER_W * SEQ,), jnp.int32),
        pltpu.VMEM((GS,), jnp.int32),
        pltpu.VMEM((GS,), jnp.int32),
        pltpu.VMEM((GS, EMBED), jnp.float32),
        pltpu.VMEM((GS, EMBED), jnp.float32),
        pltpu.VMEM((4 * 1024,), jnp.float32),
        pltpu.VMEM((4 * 1024,), jnp.float32),
        pltpu.SemaphoreType.DMA,
        pltpu.SemaphoreType.DMA,
        pltpu.SemaphoreType.DMA,
        pltpu.SemaphoreType.DMA,
        pltpu.SemaphoreType.DMA,
    ],
    compiler_params=pltpu.CompilerParams(
        use_tc_tiling_on_sc=False, needs_layout_passes=False
    ),
)
def _gather_kernel(x1d, tab, out4, xb, ib0, ib1, rv0, rv1, ab0, ab1,
                   sx, sg0, sg1, sa0, sa1):
    wid = lax.axis_index("s") * NC + lax.axis_index("c")
    b0 = wid * B_PER_W
    ibs = (ib0, ib1)
    rvs = (rv0, rv1)
    abs_ = (ab0, ab1)
    sgs = (sg0, sg1)
    sas = (sa0, sa1)

    pltpu.sync_copy(x1d.at[pl.ds(b0 * SEQ, B_PER_W * SEQ)], xb)

    iota = _iota16()
    q_pat = (iota // 8) * SEQ + (iota % 8)
    s_lo = ((iota // 8) * 1024) + ((iota % 8) * 128)
    s_hi = s_lo + 2 * 1024

    def build_and_fire(ch, slot):
        s0 = ch * SR
        ib = ibs[slot]
        def body(v, _):
            off = q_pat + (v * 400 + s0)
            idx = plsc.load_gather(xb, [off])
            ib[pl.ds(v * 16, 16)] = idx
            return 0
        lax.fori_loop(0, GS // 16, body, 0, unroll=8)
        pltpu.async_copy(tab.at[ib], rvs[slot], sgs[slot])

    def wait_gather(slot):
        pltpu.make_async_copy(tab.at[ibs[slot]], rvs[slot], sgs[slot]).wait()

    def wait_ab(slot):
        for _ in range(4):
            pltpu.make_async_copy(abs_[slot], out4.at[0, 0, :],
                                  sas[slot]).wait()

    def assemble(ch, slot):
        rv = rvs[slot]
        s0 = ch * SR

        def sipair(sp, _):
            for sb in range(2):
                si = sp * 2 + sb
                ab = abs_[sb]
                @pl.when((si >= 2) | (ch > 0))
                def _():
                    wait_ab(sb)
                def jbody(j, _):
                    k = j * SR + si
                    lo = rv[k, pl.ds(0, 16)]
                    hi = rv[k, pl.ds(16, 16)]
                    plsc.store_scatter(ab, [s_lo + j], lo)
                    plsc.store_scatter(ab, [s_hi + j], hi)
                    return 0
                lax.fori_loop(0, B_PER_W, jbody, 0, unroll=8)
                for i in range(4):
                    pltpu.async_copy(ab.at[pl.ds(i * 1024, 1024)],
                                     out4.at[s0 + si, i * 32 + wid, :],
                                     sas[sb])
            return 0

        lax.fori_loop(0, SR // 2, sipair, 0)

    build_and_fire(0, 0)

    def chpair(p, _):
        for b in range(2):
            ch = p * 2 + b
            @pl.when(ch + 1 < NCH2)
            def _():
                build_and_fire(ch + 1, 1 - b)
            @pl.when(ch < NCH2)
            def _():
                wait_gather(b)
                assemble(ch, b)
        return 0

    lax.fori_loop(0, (NCH2 + 2) // 2, chpair, 0)
    wait_ab(0)
    wait_ab(1)


def kernel(x, table):
    tt = jnp.transpose(table)
    tail = lax.slice(tt, (0, K1_MAIN), (EMBED, VOCAB))
    t128 = _relayout_kernel(tt, tail)
    t_flat = jnp.reshape(t128, (VOCAB, EMBED))
    out4 = _gather_kernel(x.reshape(N), t_flat)
    o5 = jnp.reshape(out4, (SEQ, 4, 32, 8, 128))
    out = jnp.transpose(o5, (2, 4, 0, 1, 3)).reshape(BATCH, SEQ, EMBED)
    return out


# speedup vs baseline: 1.1897x; 1.1897x over previous
import functools

import jax
import jax.numpy as jnp
from jax import lax
from jax.experimental import pallas as pl
from jax.experimental.pallas import tpu as pltpu
from jax.experimental.pallas import tpu_sc as plsc

BATCH = 4096
SEQ = 200
EMBED = 32
VOCAB = 1000000
N = BATCH * SEQ
NC = 2
NS = 16
NW = NC * NS

_mesh = plsc.VectorSubcoreMesh(core_axis_name="c", subcore_axis_name="s")


def _iota16():
    return lax.iota(jnp.int32, 16)


K1_C = 512
K1_MAIN = (VOCAB // 128) * 128
K1_CHUNKS = K1_MAIN // K1_C
K1_TAIL = VOCAB - K1_MAIN


@functools.partial(
    pl.kernel,
    mesh=_mesh,
    out_type=jax.ShapeDtypeStruct((VOCAB // 4, 128), jnp.float32),
    scratch_types=[
        pltpu.VMEM((EMBED, K1_C), jnp.float32),
        pltpu.VMEM((EMBED, K1_C), jnp.float32),
        pltpu.VMEM((K1_C // 4, 128), jnp.float32),
        pltpu.VMEM((K1_C // 4, 128), jnp.float32),
        pltpu.VMEM((EMBED, 64), jnp.float32),
        pltpu.SemaphoreType.DMA,
        pltpu.SemaphoreType.DMA,
        pltpu.SemaphoreType.DMA,
        pltpu.SemaphoreType.DMA,
    ],
    compiler_params=pltpu.CompilerParams(
        use_tc_tiling_on_sc=True, needs_layout_passes=False
    ),
)
def _relayout_kernel(tt, tail, t128, in0, in1, out0, out1, tin,
                     si0, si1, so0, so1):
    wid = lax.axis_index("s") * NC + lax.axis_index("c")
    nchunk = K1_CHUNKS // NW
    rem = K1_CHUNKS - nchunk * NW
    my_total = nchunk + jnp.where(wid < rem, 1, 0)
    e_lo = _iota16()
    e_hi = e_lo + 16

    ins = (in0, in1)
    outs = (out0, out1)
    sis = (si0, si1)
    sos = (so0, so1)

    def shuffle(in_buf, out_buf):
        def body(rr, _):
            q = rr // 4
            c0 = (rr % 4) * 32
            lo = plsc.load_gather(in_buf, [e_lo, jnp.full((16,), rr, jnp.int32)])
            hi = plsc.load_gather(in_buf, [e_hi, jnp.full((16,), rr, jnp.int32)])
            out_buf[q, pl.ds(c0, 16)] = lo
            out_buf[q, pl.ds(c0 + 16, 16)] = hi
            return 0
        lax.fori_loop(0, K1_C, body, 0, unroll=8)

    def chunk_r0(k):
        return pl.multiple_of((k * NW + wid) * K1_C, K1_C)

    def start_in(k, slot):
        return pltpu.async_copy(tt.at[:, pl.ds(chunk_r0(k), K1_C)],
                                ins[slot], sis[slot])

    def wait_in(slot):
        pltpu.make_async_copy(tt.at[:, pl.ds(0, K1_C)], ins[slot],
                              sis[slot]).wait()

    def wait_out(slot):
        pltpu.make_async_copy(outs[slot], t128.at[pl.ds(0, K1_C // 4), :],
                              sos[slot]).wait()

    start_in(0, 0)
    npair = (nchunk + 2) // 2

    def pair(p, _):
        for b in range(2):
            k = p * 2 + b
            @pl.when(k + 1 < my_total)
            def _():
                start_in(k + 1, 1 - b)
            @pl.when((k >= 2) & (k < my_total))
            def _():
                wait_out(b)
            @pl.when(k < my_total)
            def _():
                wait_in(b)
                shuffle(ins[b], outs[b])
                q0 = pl.multiple_of(chunk_r0(k) // 4, K1_C // 4)
                pltpu.async_copy(outs[b], t128.at[pl.ds(q0, K1_C // 4), :],
                                 sos[b])
        return 0

    lax.fori_loop(0, npair, pair, 0)
    wait_out(0)
    wait_out(1)

    @pl.when(wid == NW - 1)
    def _():
        pltpu.sync_copy(tail, tin)
        def body(rr, _):
            q = rr // 4
            c0 = (rr % 4) * 32
            lo = plsc.load_gather(tin, [e_lo, jnp.full((16,), rr, jnp.int32)])
            hi = plsc.load_gather(tin, [e_hi, jnp.full((16,), rr, jnp.int32)])
            out0[q, pl.ds(c0, 16)] = lo
            out0[q, pl.ds(c0 + 16, 16)] = hi
            return 0
        lax.fori_loop(0, K1_TAIL, body, 0, unroll=8)
        pltpu.sync_copy(out0.at[pl.ds(0, K1_TAIL // 4), :],
                        t128.at[pl.ds(K1_MAIN // 4, K1_TAIL // 4), :])


B_PER_W = BATCH // NW
SR = 8
NCH2 = SEQ // SR
GS = SR * B_PER_W


@functools.partial(
    pl.kernel,
    mesh=_mesh,
    out_type=jax.ShapeDtypeStruct((SEQ, 128, 1024), jnp.float32),
    scratch_types=[
        pltpu.VMEM((B_PER_W * SEQ,), jnp.int32),
        pltpu.VMEM((GS,), jnp.int32),
        pltpu.VMEM((GS,), jnp.int32),
        pltpu.VMEM((GS, EMBED), jnp.float32),
        pltpu.VMEM((GS, EMBED), jnp.float32),
        pltpu.VMEM((4 * 1024,), jnp.float32),
        pltpu.VMEM((4 * 1024,), jnp.float32),
        pltpu.SemaphoreType.DMA,
        pltpu.SemaphoreType.DMA,
        pltpu.SemaphoreType.DMA,
        pltpu.SemaphoreType.DMA,
        pltpu.SemaphoreType.DMA,
    ],
    compiler_params=pltpu.CompilerParams(
        use_tc_tiling_on_sc=False, needs_layout_passes=False
    ),
)
def _gather_kernel(x1d, tab, out4, xb, ib0, ib1, rv0, rv1, ab0, ab1,
                   sx, sg0, sg1, sa0, sa1):
    wid = lax.axis_index("s") * NC + lax.axis_index("c")
    b0 = wid * B_PER_W
    ibs = (ib0, ib1)
    rvs = (rv0, rv1)
    abs_ = (ab0, ab1)
    sgs = (sg0, sg1)
    sas = (sa0, sa1)

    pltpu.sync_copy(x1d.at[pl.ds(b0 * SEQ, B_PER_W * SEQ)], xb)

    iota = _iota16()
    q_pat = (iota // 8) * SEQ + (iota % 8)
    s_lo = ((iota // 8) * 1024) + ((iota % 8) * 128)
    s_hi = s_lo + 2 * 1024

    def build_and_fire(ch, slot):
        s0 = ch * SR
        ib = ibs[slot]
        def body(v, _):
            off = q_pat + (v * 400 + s0)
            idx = plsc.load_gather(xb, [off])
            ib[pl.ds(v * 16, 16)] = idx
            return 0
        lax.fori_loop(0, GS // 16, body, 0, unroll=8)
        pltpu.async_copy(tab.at[ib], rvs[slot], sgs[slot])

    def wait_gather(slot):
        pltpu.make_async_copy(tab.at[ibs[slot]], rvs[slot], sgs[slot]).wait()

    def wait_ab(slot):
        for _ in range(4):
            pltpu.make_async_copy(abs_[slot], out4.at[0, 0, :],
                                  sas[slot]).wait()

    def assemble(ch, slot):
        rv = rvs[slot]
        s0 = ch * SR

        def sipair(sp, _):
            for sb in range(2):
                si = sp * 2 + sb
                ab = abs_[sb]
                @pl.when((si >= 2) | (ch > 0))
                def _():
                    wait_ab(sb)
                def jbody(j, _):
                    k = j * SR + si
                    lo = rv[k, pl.ds(0, 16)]
                    hi = rv[k, pl.ds(16, 16)]
                    plsc.store_scatter(ab, [s_lo + j], lo)
                    plsc.store_scatter(ab, [s_hi + j], hi)
                    return 0
                lax.fori_loop(0, B_PER_W, jbody, 0, unroll=8)
                for i in range(4):
                    pltpu.async_copy(ab.at[pl.ds(i * 1024, 1024)],
                                     out4.at[s0 + si, i * 32 + wid, :],
                                     sas[sb])
            return 0

        lax.fori_loop(0, SR // 2, sipair, 0)

    build_and_fire(0, 0)

    def chpair(p, _):
        for b in range(2):
            ch = p * 2 + b
            @pl.when(ch + 1 < NCH2)
            def _():
                build_and_fire(ch + 1, 1 - b)
            @pl.when(ch < NCH2)
            def _():
                wait_gather(b)
                assemble(ch, b)
        return 0

    lax.fori_loop(0, (NCH2 + 2) // 2, chpair, 0)
    wait_ab(0)
    wait_ab(1)


def kernel(x, table):
    out4 = _gather_kernel(x.reshape(N), table)
    o5 = jnp.reshape(out4, (SEQ, 4, 32, 8, 128))
    out = jnp.transpose(o5, (2, 4, 0, 1, 3)).reshape(BATCH, SEQ, EMBED)
    return out
